# baseline (speedup 1.0000x reference)
import functools

import jax
import jax.numpy as jnp
from jax import lax
from jax.experimental import pallas as pl
from jax.experimental.pallas import tpu as pltpu
from jax.experimental.pallas import tpu_sc as plsc

LAMB_A = 0.7
LAMB_B = 1.0 - 0.7

N = 50000
D = 256
NW = 32
ROWS_W = 1568
C = 56
NCH = ROWS_W // C
PIECES = D // 16


@functools.partial(
    pl.kernel,
    out_type=(
        jax.ShapeDtypeStruct((N, D), jnp.float32),
        jax.ShapeDtypeStruct((N,), jnp.int32),
    ),
    mesh=plsc.VectorSubcoreMesh(core_axis_name="c", subcore_axis_name="s"),
    scratch_types=[
        pltpu.VMEM((ROWS_W,), jnp.int32),
        pltpu.VMEM((ROWS_W,), jnp.int32),
        pltpu.VMEM((C, D), jnp.float32),
        pltpu.VMEM((C, D), jnp.float32),
        pltpu.VMEM((C, D), jnp.float32),
        pltpu.VMEM((C, D), jnp.float32),
        pltpu.VMEM((C, D), jnp.float32),
        pltpu.VMEM((C, D), jnp.float32),
        pltpu.VMEM((C, D), jnp.float32),
        pltpu.VMEM((C, D), jnp.float32),
        pltpu.SemaphoreType.DMA,
        pltpu.SemaphoreType.DMA,
        pltpu.SemaphoreType.DMA,
        pltpu.SemaphoreType.DMA,
        pltpu.SemaphoreType.DMA,
        pltpu.SemaphoreType.DMA,
        pltpu.SemaphoreType.DMA,
        pltpu.SemaphoreType.DMA,
        pltpu.SemaphoreType.DMA,
    ],
)
def _mixup_kernel(x_hbm, y_hbm, pair_hbm, xmix_hbm, ynew_hbm,
                  idx_v, y_v, x0, x1, xb0, xb1, xb2, xb3, o0, o1,
                  sx0, sx1, sg0, sg1, sg2, sg3, ss0, ss1, sy):
    wid = lax.axis_index("s") * 2 + lax.axis_index("c")
    wbase = jnp.minimum(wid * ROWS_W, N - ROWS_W)

    x_v = (x0, x1)
    xb_v = (xb0, xb1, xb2, xb3)
    o_v = (o0, o1)
    sx = (sx0, sx1)
    sg = (sg0, sg1, sg2, sg3)
    ss = (ss0, ss1)

    pltpu.sync_copy(pair_hbm.at[pl.ds(wbase, ROWS_W)], idx_v)

    def fetch_x(c):
        base = wbase + c * C
        return pltpu.async_copy(x_hbm.at[pl.ds(base, C)], x_v[c % 2], sx[c % 2])

    def fetch_g(c):
        b = c % 4
        return pltpu.async_copy(
            x_hbm.at[idx_v.at[pl.ds(c * C, C)]], xb_v[b], sg[b])

    xdescs = {0: fetch_x(0), 1: fetch_x(1)}
    gdescs = {0: fetch_g(0), 1: fetch_g(1), 2: fetch_g(2)}

    dy_in = pltpu.async_copy(y_hbm.at[pl.ds(wbase, ROWS_W)], y_v, sy)
    dy_in.wait()
    dy_out = pltpu.async_copy(y_v, ynew_hbm.at[pl.ds(wbase, ROWS_W)], sy)

    store_descs = {}
    for c in range(NCH):
        b = c % 4
        bx = c % 2
        bo = c % 2
        xdescs.pop(c).wait()
        gdescs.pop(c).wait()
        if c + 3 < NCH:
            gdescs[c + 3] = fetch_g(c + 3)
        if c >= 2:
            store_descs[c - 2].wait()

        def row_body(i, _, b=b, bx=bx, bo=bo):
            for j in range(PIECES):
                sl = pl.ds(j * 16, 16)
                o_v[bo][i, sl] = LAMB_A * x_v[bx][i, sl] + LAMB_B * xb_v[b][i, sl]
            return 0

        lax.fori_loop(0, C, row_body, 0, unroll=False)

        if c + 2 < NCH:
            xdescs[c + 2] = fetch_x(c + 2)
        store_descs[c] = pltpu.async_copy(
            o_v[bo], xmix_hbm.at[pl.ds(wbase + c * C, C)], ss[bo])

    dy_out.wait()
    store_descs[NCH - 2].wait()
    store_descs[NCH - 1].wait()


def kernel(x, y, pair_idx):
    x_mix, new_y = _mixup_kernel(x, y, pair_idx)
    return (x_mix, new_y)


# speedup vs baseline: 1.1153x; 1.1153x over previous
import functools

import jax
import jax.numpy as jnp
from jax import lax
from jax.experimental import pallas as pl
from jax.experimental.pallas import tpu as pltpu
from jax.experimental.pallas import tpu_sc as plsc

LAMB_A = 0.7
LAMB_B = 1.0 - 0.7

N = 50000
D = 256
NW = 32
ROWS_W = 1584
C = 48
NCH = ROWS_W // C
NG = NCH // 3
PIECES = D // 16


@functools.partial(
    pl.kernel,
    out_type=(
        jax.ShapeDtypeStruct((N, D), jnp.float32),
        jax.ShapeDtypeStruct((N,), jnp.int32),
    ),
    mesh=plsc.VectorSubcoreMesh(core_axis_name="c", subcore_axis_name="s"),
    scratch_types=[
        pltpu.VMEM((ROWS_W,), jnp.int32),
        pltpu.VMEM((ROWS_W,), jnp.int32),
        pltpu.VMEM((C, D), jnp.float32),
        pltpu.VMEM((C, D), jnp.float32),
        pltpu.VMEM((C, D), jnp.float32),
        pltpu.VMEM((C, D), jnp.float32),
        pltpu.VMEM((C, D), jnp.float32),
        pltpu.VMEM((C, D), jnp.float32),
        pltpu.VMEM((C, D), jnp.float32),
        pltpu.VMEM((C, D), jnp.float32),
        pltpu.VMEM((C, D), jnp.float32),
        pltpu.SemaphoreType.DMA,
        pltpu.SemaphoreType.DMA,
        pltpu.SemaphoreType.DMA,
        pltpu.SemaphoreType.DMA,
        pltpu.SemaphoreType.DMA,
        pltpu.SemaphoreType.DMA,
        pltpu.SemaphoreType.DMA,
        pltpu.SemaphoreType.DMA,
        pltpu.SemaphoreType.DMA,
        pltpu.SemaphoreType.DMA,
    ],
)
def _mixup_kernel(x_hbm, y_hbm, pair_hbm, xmix_hbm, ynew_hbm,
                  idx_v, y_v, x0, x1, x2, xb0, xb1, xb2, o0, o1, o2,
                  sx0, sx1, sx2, sg0, sg1, sg2, ss0, ss1, ss2, sy):
    wid = lax.axis_index("s") * 2 + lax.axis_index("c")
    wbase = jnp.minimum(wid * ROWS_W, N - ROWS_W)

    x_v = (x0, x1, x2)
    xb_v = (xb0, xb1, xb2)
    o_v = (o0, o1, o2)
    sx = (sx0, sx1, sx2)
    sg = (sg0, sg1, sg2)
    ss = (ss0, ss1, ss2)

    pltpu.sync_copy(pair_hbm.at[pl.ds(wbase, ROWS_W)], idx_v)

    def fetch(c, b):
        base = wbase + c * C
        pltpu.async_copy(x_hbm.at[pl.ds(base, C)], x_v[b], sx[b])
        pltpu.async_copy(x_hbm.at[idx_v.at[pl.ds(c * C, C)]], xb_v[b], sg[b])

    fetch(0, 0)
    fetch(1, 1)

    pltpu.async_copy(y_hbm.at[pl.ds(wbase, ROWS_W)], y_v, sy).wait()
    dy_out = pltpu.async_copy(y_v, ynew_hbm.at[pl.ds(wbase, ROWS_W)], sy)

    def group_body(g, _):
        for b in range(3):
            c = 3 * g + b
            pltpu.make_async_copy(x_hbm.at[pl.ds(0, C)], x_v[b], sx[b]).wait()
            pltpu.make_async_copy(x_hbm.at[pl.ds(0, C)], xb_v[b], sg[b]).wait()

            @pl.when(c + 2 < NCH)
            def _(c=c, b=b):
                fetch(c + 2, (b + 2) % 3)

            @pl.when(c >= 3)
            def _(b=b):
                pltpu.make_async_copy(
                    o_v[b], xmix_hbm.at[pl.ds(wbase, C)], ss[b]).wait()

            def row_body(i, _, b=b):
                for j in range(PIECES):
                    sl = pl.ds(j * 16, 16)
                    o_v[b][i, sl] = (LAMB_A * x_v[b][i, sl]
                                     + LAMB_B * xb_v[b][i, sl])
                return 0

            lax.fori_loop(0, C, row_body, 0, unroll=False)

            pltpu.async_copy(o_v[b], xmix_hbm.at[pl.ds(wbase + c * C, C)],
                             ss[b])
        return 0

    lax.fori_loop(0, NG, group_body, 0, unroll=False)

    dy_out.wait()
    for b in range(3):
        pltpu.make_async_copy(
            o_v[b], xmix_hbm.at[pl.ds(wbase, C)], ss[b]).wait()


def kernel(x, y, pair_idx):
    x_mix, new_y = _mixup_kernel(x, y, pair_idx)
    return (x_mix, new_y)
